# baseline (speedup 1.0000x reference)
import functools

import jax
import jax.numpy as jnp
from jax import lax
from jax.experimental import pallas as pl
from jax.experimental.pallas import tpu as pltpu
from jax.experimental.pallas import tpu_sc as plsc

_N = 10000
_D = 128
_E = 320000
_NP = 10240
_NC = 2
_NS = 16
_NW = _NC * _NS
_CB = 128
_CH = -(-_E // (_NW * _CB))
_EP = _NW * _CB * _CH
_RPS = _NP // _NS
_DW = 128
_R = 1024

_CBA = 64
_CHF = 204
_CHS = 112
_FAST = 0
_EF = _CHF * _CBA
_ES = _CHS * _CBA

@functools.cache
def _sc_kernels():
    mesh = plsc.VectorSubcoreMesh(core_axis_name="c", subcore_axis_name="s")

    @functools.partial(
        pl.kernel,
        mesh=mesh,
        out_type=jax.ShapeDtypeStruct((_NC, _NP, _DW), jnp.float32),
        scratch_types=[
            pltpu.VMEM((_CH, _CB), jnp.int32),
            pltpu.VMEM((_CB, _DW), jnp.float32),
            pltpu.VMEM_SHARED((_NP, _DW), jnp.float32),
        ],
    )
    def sc_degree(dst_hbm, ones_hbm, zero_hbm, out_hbm, dst_v, ones_v, acc_sh):
        c = lax.axis_index("c")
        s = lax.axis_index("s")
        wid = s * _NC + c
        pltpu.sync_copy(dst_hbm.at[wid], dst_v)
        pltpu.sync_copy(ones_hbm, ones_v)
        pltpu.sync_copy(zero_hbm, acc_sh.at[pl.ds(s * _RPS, _RPS)])
        plsc.subcore_barrier()

        def body(j, carry):
            pltpu.sync_copy(ones_v, acc_sh.at[dst_v.at[j]], add=True)
            return carry

        lax.fori_loop(0, _CH, body, 0)
        plsc.subcore_barrier()
        pltpu.sync_copy(acc_sh.at[pl.ds(s * _RPS, _RPS)],
                        out_hbm.at[c].at[pl.ds(s * _RPS, _RPS)])

    @functools.partial(
        pl.kernel,
        mesh=mesh,
        out_type=jax.ShapeDtypeStruct((_NC, _NP, _D), jnp.float32),
        scratch_types=[
            pltpu.VMEM((2, _CBA), jnp.int32),
            pltpu.VMEM((2, _CBA), jnp.int32),
            pltpu.VMEM((_CBA, _D), jnp.float32),
            pltpu.VMEM((_CBA, _D), jnp.float32),
            pltpu.VMEM_SHARED((_NP, _D), jnp.float32),
            pltpu.SemaphoreType.DMA,
            pltpu.SemaphoreType.DMA,
            pltpu.SemaphoreType.DMA,
            pltpu.SemaphoreType.DMA,
            pltpu.SemaphoreType.DMA,
            pltpu.SemaphoreType.DMA,
        ],
    )
    def sc_aggregate(y_hbm, ei_hbm, zero_hbm, out_hbm,
                     idx0, idx1, rows0, rows1, acc_sh,
                     is0, is1, gs0, gs1, ss0, ss1):
        c = lax.axis_index("c")
        s = lax.axis_index("s")
        wid = s * _NC + c
        pltpu.sync_copy(zero_hbm, acc_sh.at[pl.ds(s * _RPS, _RPS)])
        plsc.subcore_barrier()

        nch = lax.select(c == _FAST, _CHF, _CHS)
        last = nch - 1
        pltpu.async_copy(ei_hbm.at[wid, 0], idx0, is0)
        pltpu.async_copy(ei_hbm.at[wid, 1], idx1, is1)
        pltpu.make_async_copy(ei_hbm.at[wid, 0], idx0, is0).wait()
        pltpu.async_copy(y_hbm.at[idx0.at[0]], rows0, gs0)

        def body(j, carry):
            e0 = 2 * j
            e1 = 2 * j + 1
            pltpu.make_async_copy(y_hbm.at[idx0.at[0]], rows0, gs0).wait()
            pltpu.async_copy(rows0, acc_sh.at[idx0.at[1]], ss0, add=True)
            pltpu.make_async_copy(ei_hbm.at[wid, e1], idx1, is1).wait()
            pltpu.async_copy(y_hbm.at[idx1.at[0]], rows1, gs1)
            pltpu.make_async_copy(rows0, acc_sh.at[idx0.at[1]], ss0).wait()
            pltpu.async_copy(
                ei_hbm.at[wid, jnp.minimum(e0 + 2, last)], idx0, is0)
            pltpu.make_async_copy(y_hbm.at[idx1.at[0]], rows1, gs1).wait()
            pltpu.async_copy(rows1, acc_sh.at[idx1.at[1]], ss1, add=True)
            pltpu.make_async_copy(ei_hbm.at[wid, e0], idx0, is0).wait()
            pltpu.async_copy(y_hbm.at[idx0.at[0]], rows0, gs0)
            pltpu.make_async_copy(rows1, acc_sh.at[idx1.at[1]], ss1).wait()
            pltpu.async_copy(
                ei_hbm.at[wid, jnp.minimum(e1 + 2, last)], idx1, is1)
            return carry

        lax.fori_loop(0, nch // 2 - 1, body, 0)
        pltpu.make_async_copy(y_hbm.at[idx0.at[0]], rows0, gs0).wait()
        pltpu.sync_copy(rows0, acc_sh.at[idx0.at[1]], add=True)
        pltpu.make_async_copy(ei_hbm.at[wid, 0], idx1, is1).wait()
        pltpu.sync_copy(y_hbm.at[idx1.at[0]], rows1)
        pltpu.sync_copy(rows1, acc_sh.at[idx1.at[1]], add=True)
        plsc.subcore_barrier()
        pltpu.sync_copy(acc_sh.at[pl.ds(s * _RPS, _RPS)],
                        out_hbm.at[c].at[pl.ds(s * _RPS, _RPS)])

    return sc_degree, sc_aggregate


def _tc_prep(degp, x_p, W):
    def body(deg_ref, x_ref, w_ref, y_ref, xw_ref, dinv_ref):
        deg = deg_ref[0] + deg_ref[1] + 1.0
        dinv = lax.rsqrt(deg)
        xw = jnp.dot(x_ref[...], w_ref[...], preferred_element_type=jnp.float32)
        y_ref[...] = dinv * xw
        xw_ref[...] = xw
        dinv_ref[...] = dinv

    return pl.pallas_call(
        body,
        grid=(_NP // _R,),
        in_specs=[
            pl.BlockSpec((_NC, _R, _DW), lambda i: (0, i, 0)),
            pl.BlockSpec((_R, _D), lambda i: (i, 0)),
            pl.BlockSpec((_D, _D), lambda i: (0, 0)),
        ],
        out_specs=[
            pl.BlockSpec((_R, _D), lambda i: (i, 0)),
            pl.BlockSpec((_R, _D), lambda i: (i, 0)),
            pl.BlockSpec((_R, _DW), lambda i: (i, 0)),
        ],
        out_shape=[
            jax.ShapeDtypeStruct((_NP, _D), jnp.float32),
            jax.ShapeDtypeStruct((_NP, _D), jnp.float32),
            jax.ShapeDtypeStruct((_NP, _DW), jnp.float32),
        ],
    )(degp, x_p, W)


def _tc_mid(parts, xw1, dinv, b1, W2):
    def body(p_ref, xw_ref, dinv_ref, b_ref, w_ref, y_ref, xw2_ref):
        dv = dinv_ref[...]
        h = dv * (p_ref[0] + p_ref[1]) + (dv * dv) * xw_ref[...] + b_ref[...]
        h = jnp.maximum(h, 0.0)
        xw2 = jnp.dot(h, w_ref[...], preferred_element_type=jnp.float32)
        y_ref[...] = dv * xw2
        xw2_ref[...] = xw2

    return pl.pallas_call(
        body,
        grid=(_NP // _R,),
        in_specs=[
            pl.BlockSpec((_NC, _R, _D), lambda i: (0, i, 0)),
            pl.BlockSpec((_R, _D), lambda i: (i, 0)),
            pl.BlockSpec((_R, _DW), lambda i: (i, 0)),
            pl.BlockSpec((1, _D), lambda i: (0, 0)),
            pl.BlockSpec((_D, _D), lambda i: (0, 0)),
        ],
        out_specs=[
            pl.BlockSpec((_R, _D), lambda i: (i, 0)),
            pl.BlockSpec((_R, _D), lambda i: (i, 0)),
        ],
        out_shape=[
            jax.ShapeDtypeStruct((_NP, _D), jnp.float32),
            jax.ShapeDtypeStruct((_NP, _D), jnp.float32),
        ],
    )(parts, xw1, dinv, b1, W2)


def _tc_final(parts, xw2, dinv, b2):
    def body(p_ref, xw_ref, dinv_ref, b_ref, o_ref):
        dv = dinv_ref[...]
        o_ref[...] = dv * (p_ref[0] + p_ref[1]) + (dv * dv) * xw_ref[...] + b_ref[...]

    return pl.pallas_call(
        body,
        grid=(_NP // _R,),
        in_specs=[
            pl.BlockSpec((_NC, _R, _D), lambda i: (0, i, 0)),
            pl.BlockSpec((_R, _D), lambda i: (i, 0)),
            pl.BlockSpec((_R, _DW), lambda i: (i, 0)),
            pl.BlockSpec((1, _D), lambda i: (0, 0)),
        ],
        out_specs=pl.BlockSpec((_R, _D), lambda i: (i, 0)),
        out_shape=jax.ShapeDtypeStruct((_NP, _D), jnp.float32),
    )(parts, xw2, dinv, b2)


def _split_edges(v):
    esr = (_E - (_NW // _NC) * _EF) // (_NW // _NC)
    slow_base = (_NW // _NC) * _EF
    rows = []
    for wid in range(_NW):
        k = wid // _NC
        if wid % _NC == _FAST:
            r = v[k * _EF:(k + 1) * _EF]
        else:
            r = v[slow_base + k * esr: slow_base + (k + 1) * esr]
            r = jnp.concatenate([r, jnp.full((_EF - esr,), _N, jnp.int32)])
        rows.append(r)
    return jnp.stack(rows).reshape(_NW, _CHF, _CBA)


def kernel(x, edge_index, W1, b1, W2, b2):
    src = edge_index[0].astype(jnp.int32)
    dst = edge_index[1].astype(jnp.int32)
    pad = jnp.full((_EP - _E,), _N, dtype=jnp.int32)
    dst_deg = jnp.concatenate([dst, pad]).reshape(_NW, _CH, _CB)
    ei_a = jnp.stack([_split_edges(src), _split_edges(dst)], axis=2)
    x_p = jnp.pad(x, ((0, _NP - _N), (0, 0)))
    ones_dw = jnp.ones((_CB, _DW), jnp.float32)
    zero_d = jnp.zeros((_RPS, _D), jnp.float32)

    sc_degree, sc_aggregate = _sc_kernels()
    degp = sc_degree(dst_deg, ones_dw, zero_d)
    y1, xw1, dinv = _tc_prep(degp, x_p, W1)
    p1 = sc_aggregate(y1, ei_a, zero_d)
    y2, xw2 = _tc_mid(p1, xw1, dinv, b1.reshape(1, _D), W2)
    p2 = sc_aggregate(y2, ei_a, zero_d)
    out = _tc_final(p2, xw2, dinv, b2.reshape(1, _D))
    return out[:_N]


# speedup vs baseline: 1.1154x; 1.1154x over previous
import functools

import jax
import jax.numpy as jnp
from jax import lax
from jax.experimental import pallas as pl
from jax.experimental.pallas import tpu as pltpu
from jax.experimental.pallas import tpu_sc as plsc

_N = 10000
_D = 128
_E = 320000
_NP = 10240
_NC = 2
_NS = 16
_NW = _NC * _NS
_CBD = 64
_CHD = 158
_EP = _NW * _CBD * _CHD
_RPS = _NP // _NS
_DW = 128
_R = 1024

_CBA = 128
_CHF = 102
_CHS = 56
_FAST = 0
_EF = _CHF * _CBA
_ES = _CHS * _CBA

@functools.cache
def _sc_kernels():
    mesh = plsc.VectorSubcoreMesh(core_axis_name="c", subcore_axis_name="s")

    @functools.partial(
        pl.kernel,
        mesh=mesh,
        out_type=jax.ShapeDtypeStruct((_NC, _NP, _DW), jnp.float32),
        scratch_types=[
            pltpu.VMEM((1, _CBD), jnp.int32),
            pltpu.VMEM((1, _CBD), jnp.int32),
            pltpu.VMEM((_CBD, _DW), jnp.float32),
            pltpu.VMEM_SHARED((_NP, _DW), jnp.float32),
            pltpu.SemaphoreType.DMA,
            pltpu.SemaphoreType.DMA,
            pltpu.SemaphoreType.DMA,
            pltpu.SemaphoreType.DMA,
        ],
    )
    def sc_degree(dst_hbm, ones_hbm, zero_hbm, out_hbm,
                  idx0, idx1, ones_v, acc_sh, is0, is1, ss0, ss1):
        c = lax.axis_index("c")
        s = lax.axis_index("s")
        wid = s * _NC + c
        pltpu.sync_copy(ones_hbm, ones_v)
        pltpu.sync_copy(zero_hbm, acc_sh.at[pl.ds(s * _RPS, _RPS)])
        plsc.subcore_barrier()

        pltpu.async_copy(dst_hbm.at[wid, 0], idx0, is0)
        pltpu.async_copy(dst_hbm.at[wid, 1], idx1, is1)

        def body(j, carry):
            e0 = 2 * j
            e1 = 2 * j + 1
            pltpu.make_async_copy(dst_hbm.at[wid, e0], idx0, is0).wait()
            pltpu.async_copy(ones_v, acc_sh.at[idx0.at[0]], ss0, add=True)
            pltpu.make_async_copy(dst_hbm.at[wid, e1], idx1, is1).wait()
            pltpu.async_copy(ones_v, acc_sh.at[idx1.at[0]], ss1, add=True)
            pltpu.make_async_copy(ones_v, acc_sh.at[idx0.at[0]], ss0).wait()
            pltpu.async_copy(dst_hbm.at[wid, e0 + 2], idx0, is0)
            pltpu.make_async_copy(ones_v, acc_sh.at[idx1.at[0]], ss1).wait()
            pltpu.async_copy(dst_hbm.at[wid, e1 + 2], idx1, is1)
            return carry

        lax.fori_loop(0, _CHD // 2 - 1, body, 0)
        pltpu.make_async_copy(dst_hbm.at[wid, 0], idx0, is0).wait()
        pltpu.sync_copy(ones_v, acc_sh.at[idx0.at[0]], add=True)
        pltpu.make_async_copy(dst_hbm.at[wid, 0], idx1, is1).wait()
        pltpu.sync_copy(ones_v, acc_sh.at[idx1.at[0]], add=True)
        plsc.subcore_barrier()
        pltpu.sync_copy(acc_sh.at[pl.ds(s * _RPS, _RPS)],
                        out_hbm.at[c].at[pl.ds(s * _RPS, _RPS)])

    @functools.partial(
        pl.kernel,
        mesh=mesh,
        out_type=jax.ShapeDtypeStruct((_NC, _NP, _D), jnp.float32),
        scratch_types=[
            pltpu.VMEM((2, _CBA), jnp.int32),
            pltpu.VMEM((2, _CBA), jnp.int32),
            pltpu.VMEM((_CBA, _D), jnp.float32),
            pltpu.VMEM((_CBA, _D), jnp.float32),
            pltpu.VMEM_SHARED((_NP, _D), jnp.float32),
            pltpu.SemaphoreType.DMA,
            pltpu.SemaphoreType.DMA,
            pltpu.SemaphoreType.DMA,
            pltpu.SemaphoreType.DMA,
            pltpu.SemaphoreType.DMA,
            pltpu.SemaphoreType.DMA,
        ],
    )
    def sc_aggregate(y_hbm, ei_hbm, zero_hbm, out_hbm,
                     idx0, idx1, rows0, rows1, acc_sh,
                     is0, is1, gs0, gs1, ss0, ss1):
        c = lax.axis_index("c")
        s = lax.axis_index("s")
        wid = s * _NC + c
        pltpu.sync_copy(zero_hbm, acc_sh.at[pl.ds(s * _RPS, _RPS)])
        plsc.subcore_barrier()

        nch = lax.select(c == _FAST, _CHF, _CHS)
        last = nch - 1
        pltpu.async_copy(ei_hbm.at[wid, 0], idx0, is0)
        pltpu.async_copy(ei_hbm.at[wid, 1], idx1, is1)
        pltpu.make_async_copy(ei_hbm.at[wid, 0], idx0, is0).wait()
        pltpu.async_copy(y_hbm.at[idx0.at[0]], rows0, gs0)

        def body(j, carry):
            e0 = 2 * j
            e1 = 2 * j + 1
            pltpu.make_async_copy(y_hbm.at[idx0.at[0]], rows0, gs0).wait()
            pltpu.async_copy(rows0, acc_sh.at[idx0.at[1]], ss0, add=True)
            pltpu.make_async_copy(ei_hbm.at[wid, e1], idx1, is1).wait()
            pltpu.async_copy(y_hbm.at[idx1.at[0]], rows1, gs1)
            pltpu.make_async_copy(rows0, acc_sh.at[idx0.at[1]], ss0).wait()
            pltpu.async_copy(
                ei_hbm.at[wid, jnp.minimum(e0 + 2, last)], idx0, is0)
            pltpu.make_async_copy(y_hbm.at[idx1.at[0]], rows1, gs1).wait()
            pltpu.async_copy(rows1, acc_sh.at[idx1.at[1]], ss1, add=True)
            pltpu.make_async_copy(ei_hbm.at[wid, e0], idx0, is0).wait()
            pltpu.async_copy(y_hbm.at[idx0.at[0]], rows0, gs0)
            pltpu.make_async_copy(rows1, acc_sh.at[idx1.at[1]], ss1).wait()
            pltpu.async_copy(
                ei_hbm.at[wid, jnp.minimum(e1 + 2, last)], idx1, is1)
            return carry

        lax.fori_loop(0, nch // 2 - 1, body, 0)
        pltpu.make_async_copy(y_hbm.at[idx0.at[0]], rows0, gs0).wait()
        pltpu.sync_copy(rows0, acc_sh.at[idx0.at[1]], add=True)
        pltpu.make_async_copy(ei_hbm.at[wid, 0], idx1, is1).wait()
        pltpu.sync_copy(y_hbm.at[idx1.at[0]], rows1)
        pltpu.sync_copy(rows1, acc_sh.at[idx1.at[1]], add=True)
        plsc.subcore_barrier()
        pltpu.sync_copy(acc_sh.at[pl.ds(s * _RPS, _RPS)],
                        out_hbm.at[c].at[pl.ds(s * _RPS, _RPS)])

    return sc_degree, sc_aggregate


def _tc_prep(degp, x_p, W):
    def body(deg_ref, x_ref, w_ref, y_ref, xw_ref, dinv_ref):
        deg = deg_ref[0] + deg_ref[1] + 1.0
        dinv = lax.rsqrt(deg)
        xw = jnp.dot(x_ref[...], w_ref[...], preferred_element_type=jnp.float32)
        y_ref[...] = dinv * xw
        xw_ref[...] = xw
        dinv_ref[...] = dinv

    return pl.pallas_call(
        body,
        grid=(_NP // _R,),
        in_specs=[
            pl.BlockSpec((_NC, _R, _DW), lambda i: (0, i, 0)),
            pl.BlockSpec((_R, _D), lambda i: (i, 0)),
            pl.BlockSpec((_D, _D), lambda i: (0, 0)),
        ],
        out_specs=[
            pl.BlockSpec((_R, _D), lambda i: (i, 0)),
            pl.BlockSpec((_R, _D), lambda i: (i, 0)),
            pl.BlockSpec((_R, _DW), lambda i: (i, 0)),
        ],
        out_shape=[
            jax.ShapeDtypeStruct((_NP, _D), jnp.float32),
            jax.ShapeDtypeStruct((_NP, _D), jnp.float32),
            jax.ShapeDtypeStruct((_NP, _DW), jnp.float32),
        ],
    )(degp, x_p, W)


def _tc_mid(parts, xw1, dinv, b1, W2):
    def body(p_ref, xw_ref, dinv_ref, b_ref, w_ref, y_ref, xw2_ref):
        dv = dinv_ref[...]
        h = dv * (p_ref[0] + p_ref[1]) + (dv * dv) * xw_ref[...] + b_ref[...]
        h = jnp.maximum(h, 0.0)
        xw2 = jnp.dot(h, w_ref[...], preferred_element_type=jnp.float32)
        y_ref[...] = dv * xw2
        xw2_ref[...] = xw2

    return pl.pallas_call(
        body,
        grid=(_NP // _R,),
        in_specs=[
            pl.BlockSpec((_NC, _R, _D), lambda i: (0, i, 0)),
            pl.BlockSpec((_R, _D), lambda i: (i, 0)),
            pl.BlockSpec((_R, _DW), lambda i: (i, 0)),
            pl.BlockSpec((1, _D), lambda i: (0, 0)),
            pl.BlockSpec((_D, _D), lambda i: (0, 0)),
        ],
        out_specs=[
            pl.BlockSpec((_R, _D), lambda i: (i, 0)),
            pl.BlockSpec((_R, _D), lambda i: (i, 0)),
        ],
        out_shape=[
            jax.ShapeDtypeStruct((_NP, _D), jnp.float32),
            jax.ShapeDtypeStruct((_NP, _D), jnp.float32),
        ],
    )(parts, xw1, dinv, b1, W2)


def _tc_final(parts, xw2, dinv, b2):
    def body(p_ref, xw_ref, dinv_ref, b_ref, o_ref):
        dv = dinv_ref[...]
        o_ref[...] = dv * (p_ref[0] + p_ref[1]) + (dv * dv) * xw_ref[...] + b_ref[...]

    return pl.pallas_call(
        body,
        grid=(_NP // _R,),
        in_specs=[
            pl.BlockSpec((_NC, _R, _D), lambda i: (0, i, 0)),
            pl.BlockSpec((_R, _D), lambda i: (i, 0)),
            pl.BlockSpec((_R, _DW), lambda i: (i, 0)),
            pl.BlockSpec((1, _D), lambda i: (0, 0)),
        ],
        out_specs=pl.BlockSpec((_R, _D), lambda i: (i, 0)),
        out_shape=jax.ShapeDtypeStruct((_NP, _D), jnp.float32),
    )(parts, xw2, dinv, b2)


def _split_edges(v):
    esr = (_E - (_NW // _NC) * _EF) // (_NW // _NC)
    slow_base = (_NW // _NC) * _EF
    rows = []
    for wid in range(_NW):
        k = wid // _NC
        if wid % _NC == _FAST:
            r = v[k * _EF:(k + 1) * _EF]
        else:
            r = v[slow_base + k * esr: slow_base + (k + 1) * esr]
            r = jnp.concatenate([r, jnp.full((_EF - esr,), _N, jnp.int32)])
        rows.append(r)
    return jnp.stack(rows).reshape(_NW, _CHF, _CBA)


def kernel(x, edge_index, W1, b1, W2, b2):
    src = edge_index[0].astype(jnp.int32)
    dst = edge_index[1].astype(jnp.int32)
    pad = jnp.full((_EP - _E,), _N, dtype=jnp.int32)
    dst_deg = jnp.concatenate([dst, pad]).reshape(_NW, _CHD, 1, _CBD)
    ei_a = jnp.stack([_split_edges(src), _split_edges(dst)], axis=2)
    x_p = jnp.pad(x, ((0, _NP - _N), (0, 0)))
    ones_dw = jnp.ones((_CBD, _DW), jnp.float32)
    zero_d = jnp.zeros((_RPS, _D), jnp.float32)

    sc_degree, sc_aggregate = _sc_kernels()
    degp = sc_degree(dst_deg, ones_dw, zero_d)
    y1, xw1, dinv = _tc_prep(degp, x_p, W1)
    p1 = sc_aggregate(y1, ei_a, zero_d)
    y2, xw2 = _tc_mid(p1, xw1, dinv, b1.reshape(1, _D), W2)
    p2 = sc_aggregate(y2, ei_a, zero_d)
    out = _tc_final(p2, xw2, dinv, b2.reshape(1, _D))
    return out[:_N]


# speedup vs baseline: 1.1156x; 1.0002x over previous
import functools

import jax
import jax.numpy as jnp
from jax import lax
from jax.experimental import pallas as pl
from jax.experimental.pallas import tpu as pltpu
from jax.experimental.pallas import tpu_sc as plsc

_N = 10000
_D = 128
_E = 320000
_NP = 10240
_NC = 2
_NS = 16
_NW = _NC * _NS
_CBD = 64
_CHD = 158
_EP = _NW * _CBD * _CHD
_RPS = _NP // _NS
_DW = 128
_R = 1024

_CBA = 128
_CHF = 102
_CHS = 56
_FAST = 0
_EF = _CHF * _CBA
_ES = _CHS * _CBA

@functools.cache
def _sc_kernels():
    mesh = plsc.VectorSubcoreMesh(core_axis_name="c", subcore_axis_name="s")

    @functools.partial(
        pl.kernel,
        mesh=mesh,
        out_type=jax.ShapeDtypeStruct((_NC, _NP, _DW), jnp.float32),
        scratch_types=[
            pltpu.VMEM((1, _CBD), jnp.int32),
            pltpu.VMEM((1, _CBD), jnp.int32),
            pltpu.VMEM((_CBD, _DW), jnp.float32),
            pltpu.VMEM_SHARED((_NP, _DW), jnp.float32),
            pltpu.SemaphoreType.DMA,
            pltpu.SemaphoreType.DMA,
            pltpu.SemaphoreType.DMA,
            pltpu.SemaphoreType.DMA,
        ],
    )
    def sc_degree(dst_hbm, ones_hbm, zero_hbm, out_hbm,
                  idx0, idx1, ones_v, acc_sh, is0, is1, ss0, ss1):
        c = lax.axis_index("c")
        s = lax.axis_index("s")
        wid = s * _NC + c
        pltpu.sync_copy(ones_hbm, ones_v)
        pltpu.sync_copy(zero_hbm, acc_sh.at[pl.ds(s * _RPS, _RPS)])
        plsc.subcore_barrier()

        pltpu.async_copy(dst_hbm.at[wid, 0], idx0, is0)
        pltpu.async_copy(dst_hbm.at[wid, 1], idx1, is1)

        def body(j, carry):
            e0 = 2 * j
            e1 = 2 * j + 1
            pltpu.make_async_copy(dst_hbm.at[wid, e0], idx0, is0).wait()
            pltpu.async_copy(ones_v, acc_sh.at[idx0.at[0]], ss0, add=True)
            pltpu.make_async_copy(dst_hbm.at[wid, e1], idx1, is1).wait()
            pltpu.async_copy(ones_v, acc_sh.at[idx1.at[0]], ss1, add=True)
            pltpu.make_async_copy(ones_v, acc_sh.at[idx0.at[0]], ss0).wait()
            pltpu.async_copy(dst_hbm.at[wid, e0 + 2], idx0, is0)
            pltpu.make_async_copy(ones_v, acc_sh.at[idx1.at[0]], ss1).wait()
            pltpu.async_copy(dst_hbm.at[wid, e1 + 2], idx1, is1)
            return carry

        lax.fori_loop(0, _CHD // 2 - 1, body, 0)
        pltpu.make_async_copy(dst_hbm.at[wid, 0], idx0, is0).wait()
        pltpu.sync_copy(ones_v, acc_sh.at[idx0.at[0]], add=True)
        pltpu.make_async_copy(dst_hbm.at[wid, 0], idx1, is1).wait()
        pltpu.sync_copy(ones_v, acc_sh.at[idx1.at[0]], add=True)
        plsc.subcore_barrier()
        pltpu.sync_copy(acc_sh.at[pl.ds(s * _RPS, _RPS)],
                        out_hbm.at[c].at[pl.ds(s * _RPS, _RPS)])

    @functools.partial(
        pl.kernel,
        mesh=mesh,
        out_type=jax.ShapeDtypeStruct((_NC, _NP, _D), jnp.float32),
        scratch_types=[
            pltpu.VMEM((2, _CBA), jnp.int32),
            pltpu.VMEM((2, _CBA), jnp.int32),
            pltpu.VMEM((_CBA, _D), jnp.float32),
            pltpu.VMEM((_CBA, _D), jnp.float32),
            pltpu.VMEM_SHARED((_NP, _D), jnp.float32),
            pltpu.SemaphoreType.DMA,
            pltpu.SemaphoreType.DMA,
            pltpu.SemaphoreType.DMA,
            pltpu.SemaphoreType.DMA,
            pltpu.SemaphoreType.DMA,
            pltpu.SemaphoreType.DMA,
        ],
    )
    def sc_aggregate(y_hbm, ei_hbm, zero_hbm, out_hbm,
                     idx0, idx1, rows0, rows1, acc_sh,
                     is0, is1, gs0, gs1, ss0, ss1):
        c = lax.axis_index("c")
        s = lax.axis_index("s")
        wid = s * _NC + c
        pltpu.sync_copy(zero_hbm, acc_sh.at[pl.ds(s * _RPS, _RPS)])
        plsc.subcore_barrier()

        nch = lax.select(c == _FAST, _CHF, _CHS)
        last = nch - 1
        pltpu.async_copy(ei_hbm.at[wid, 0], idx0, is0)
        pltpu.async_copy(ei_hbm.at[wid, 1], idx1, is1)
        pltpu.make_async_copy(ei_hbm.at[wid, 0], idx0, is0).wait()
        pltpu.async_copy(y_hbm.at[idx0.at[0]], rows0, gs0)

        def body(j, carry):
            e0 = 2 * j
            e1 = 2 * j + 1
            pltpu.make_async_copy(y_hbm.at[idx0.at[0]], rows0, gs0).wait()
            pltpu.async_copy(rows0, acc_sh.at[idx0.at[1]], ss0, add=True)
            pltpu.make_async_copy(ei_hbm.at[wid, e1], idx1, is1).wait()
            pltpu.async_copy(y_hbm.at[idx1.at[0]], rows1, gs1)
            pltpu.make_async_copy(rows0, acc_sh.at[idx0.at[1]], ss0).wait()
            pltpu.async_copy(
                ei_hbm.at[wid, jnp.minimum(e0 + 2, last)], idx0, is0)
            pltpu.make_async_copy(y_hbm.at[idx1.at[0]], rows1, gs1).wait()
            pltpu.async_copy(rows1, acc_sh.at[idx1.at[1]], ss1, add=True)
            pltpu.make_async_copy(ei_hbm.at[wid, e0], idx0, is0).wait()
            pltpu.async_copy(y_hbm.at[idx0.at[0]], rows0, gs0)
            pltpu.make_async_copy(rows1, acc_sh.at[idx1.at[1]], ss1).wait()
            pltpu.async_copy(
                ei_hbm.at[wid, jnp.minimum(e1 + 2, last)], idx1, is1)
            return carry

        lax.fori_loop(0, nch // 2 - 1, body, 0)
        pltpu.make_async_copy(y_hbm.at[idx0.at[0]], rows0, gs0).wait()
        pltpu.sync_copy(rows0, acc_sh.at[idx0.at[1]], add=True)
        pltpu.make_async_copy(ei_hbm.at[wid, 0], idx1, is1).wait()
        pltpu.sync_copy(y_hbm.at[idx1.at[0]], rows1)
        pltpu.sync_copy(rows1, acc_sh.at[idx1.at[1]], add=True)
        plsc.subcore_barrier()
        pltpu.sync_copy(acc_sh.at[pl.ds(s * _RPS, _RPS)],
                        out_hbm.at[c].at[pl.ds(s * _RPS, _RPS)])

    return sc_degree, sc_aggregate


def _tc_matmul1(x_p, W):
    def body(x_ref, w_ref, xw_ref):
        xw_ref[...] = jnp.dot(x_ref[...], w_ref[...],
                              preferred_element_type=jnp.float32)

    return pl.pallas_call(
        body,
        grid=(_NP // _R,),
        in_specs=[
            pl.BlockSpec((_R, _D), lambda i: (i, 0)),
            pl.BlockSpec((_D, _D), lambda i: (0, 0)),
        ],
        out_specs=pl.BlockSpec((_R, _D), lambda i: (i, 0)),
        out_shape=jax.ShapeDtypeStruct((_NP, _D), jnp.float32),
    )(x_p, W)


def _tc_scale(degp, xw):
    def body(deg_ref, xw_ref, y_ref, dinv_ref):
        deg = deg_ref[0] + deg_ref[1] + 1.0
        dinv = lax.rsqrt(deg)
        y_ref[...] = dinv * xw_ref[...]
        dinv_ref[...] = dinv

    return pl.pallas_call(
        body,
        grid=(_NP // _R,),
        in_specs=[
            pl.BlockSpec((_NC, _R, _DW), lambda i: (0, i, 0)),
            pl.BlockSpec((_R, _D), lambda i: (i, 0)),
        ],
        out_specs=[
            pl.BlockSpec((_R, _D), lambda i: (i, 0)),
            pl.BlockSpec((_R, _DW), lambda i: (i, 0)),
        ],
        out_shape=[
            jax.ShapeDtypeStruct((_NP, _D), jnp.float32),
            jax.ShapeDtypeStruct((_NP, _DW), jnp.float32),
        ],
    )(degp, xw)


def _tc_mid(parts, xw1, dinv, b1, W2):
    def body(p_ref, xw_ref, dinv_ref, b_ref, w_ref, y_ref, xw2_ref):
        dv = dinv_ref[...]
        h = dv * (p_ref[0] + p_ref[1]) + (dv * dv) * xw_ref[...] + b_ref[...]
        h = jnp.maximum(h, 0.0)
        xw2 = jnp.dot(h, w_ref[...], preferred_element_type=jnp.float32)
        y_ref[...] = dv * xw2
        xw2_ref[...] = xw2

    return pl.pallas_call(
        body,
        grid=(_NP // _R,),
        in_specs=[
            pl.BlockSpec((_NC, _R, _D), lambda i: (0, i, 0)),
            pl.BlockSpec((_R, _D), lambda i: (i, 0)),
            pl.BlockSpec((_R, _DW), lambda i: (i, 0)),
            pl.BlockSpec((1, _D), lambda i: (0, 0)),
            pl.BlockSpec((_D, _D), lambda i: (0, 0)),
        ],
        out_specs=[
            pl.BlockSpec((_R, _D), lambda i: (i, 0)),
            pl.BlockSpec((_R, _D), lambda i: (i, 0)),
        ],
        out_shape=[
            jax.ShapeDtypeStruct((_NP, _D), jnp.float32),
            jax.ShapeDtypeStruct((_NP, _D), jnp.float32),
        ],
    )(parts, xw1, dinv, b1, W2)


def _tc_final(parts, xw2, dinv, b2):
    def body(p_ref, xw_ref, dinv_ref, b_ref, o_ref):
        dv = dinv_ref[...]
        o_ref[...] = dv * (p_ref[0] + p_ref[1]) + (dv * dv) * xw_ref[...] + b_ref[...]

    return pl.pallas_call(
        body,
        grid=(_NP // _R,),
        in_specs=[
            pl.BlockSpec((_NC, _R, _D), lambda i: (0, i, 0)),
            pl.BlockSpec((_R, _D), lambda i: (i, 0)),
            pl.BlockSpec((_R, _DW), lambda i: (i, 0)),
            pl.BlockSpec((1, _D), lambda i: (0, 0)),
        ],
        out_specs=pl.BlockSpec((_R, _D), lambda i: (i, 0)),
        out_shape=jax.ShapeDtypeStruct((_NP, _D), jnp.float32),
    )(parts, xw2, dinv, b2)


def _split_edges(v):
    esr = (_E - (_NW // _NC) * _EF) // (_NW // _NC)
    slow_base = (_NW // _NC) * _EF
    rows = []
    for wid in range(_NW):
        k = wid // _NC
        if wid % _NC == _FAST:
            r = v[k * _EF:(k + 1) * _EF]
        else:
            r = v[slow_base + k * esr: slow_base + (k + 1) * esr]
            r = jnp.concatenate([r, jnp.full((_EF - esr,), _N, jnp.int32)])
        rows.append(r)
    return jnp.stack(rows).reshape(_NW, _CHF, _CBA)


def kernel(x, edge_index, W1, b1, W2, b2):
    src = edge_index[0].astype(jnp.int32)
    dst = edge_index[1].astype(jnp.int32)
    pad = jnp.full((_EP - _E,), _N, dtype=jnp.int32)
    dst_deg = jnp.concatenate([dst, pad]).reshape(_NW, _CHD, 1, _CBD)
    ei_a = jnp.stack([_split_edges(src), _split_edges(dst)], axis=2)
    x_p = jnp.pad(x, ((0, _NP - _N), (0, 0)))
    ones_dw = jnp.ones((_CBD, _DW), jnp.float32)
    zero_d = jnp.zeros((_RPS, _D), jnp.float32)

    sc_degree, sc_aggregate = _sc_kernels()
    degp = sc_degree(dst_deg, ones_dw, zero_d)
    xw1 = _tc_matmul1(x_p, W1)
    y1, dinv = _tc_scale(degp, xw1)
    p1 = sc_aggregate(y1, ei_a, zero_d)
    y2, xw2 = _tc_mid(p1, xw1, dinv, b1.reshape(1, _D), W2)
    p2 = sc_aggregate(y2, ei_a, zero_d)
    out = _tc_final(p2, xw2, dinv, b2.reshape(1, _D))
    return out[:_N]


# speedup vs baseline: 1.3347x; 1.1964x over previous
import functools

import jax
import jax.numpy as jnp
from jax import lax
from jax.experimental import pallas as pl
from jax.experimental.pallas import tpu as pltpu
from jax.experimental.pallas import tpu_sc as plsc

_N = 10000
_D = 128
_E = 320000
_NP = 10240
_NC = 2
_NS = 16
_NW = _NC * _NS
_CBD = 64
_CHD = 158
_EP = _NW * _CBD * _CHD
_RPS = _NP // _NS
_DW = 128
_R = 1024

_CBA = 96
_CHF = 138
_CHS = 72
_FAST = 0
_EF = _CHF * _CBA
_ES = _CHS * _CBA

@functools.cache
def _sc_kernels():
    mesh = plsc.VectorSubcoreMesh(core_axis_name="c", subcore_axis_name="s")

    @functools.partial(
        pl.kernel,
        mesh=mesh,
        out_type=jax.ShapeDtypeStruct((_NC, _NP, _DW), jnp.float32),
        scratch_types=[
            pltpu.VMEM((1, _CBD), jnp.int32),
            pltpu.VMEM((1, _CBD), jnp.int32),
            pltpu.VMEM((_CBD, _DW), jnp.float32),
            pltpu.VMEM_SHARED((_NP, _DW), jnp.float32),
            pltpu.SemaphoreType.DMA,
            pltpu.SemaphoreType.DMA,
            pltpu.SemaphoreType.DMA,
            pltpu.SemaphoreType.DMA,
        ],
    )
    def sc_degree(dst_hbm, ones_hbm, zero_hbm, out_hbm,
                  idx0, idx1, ones_v, acc_sh, is0, is1, ss0, ss1):
        c = lax.axis_index("c")
        s = lax.axis_index("s")
        wid = s * _NC + c
        pltpu.sync_copy(ones_hbm, ones_v)
        pltpu.sync_copy(zero_hbm, acc_sh.at[pl.ds(s * _RPS, _RPS)])
        plsc.subcore_barrier()

        pltpu.async_copy(dst_hbm.at[wid, 0], idx0, is0)
        pltpu.async_copy(dst_hbm.at[wid, 1], idx1, is1)

        def body(j, carry):
            e0 = 2 * j
            e1 = 2 * j + 1
            pltpu.make_async_copy(dst_hbm.at[wid, e0], idx0, is0).wait()
            pltpu.async_copy(ones_v, acc_sh.at[idx0.at[0]], ss0, add=True)
            pltpu.make_async_copy(dst_hbm.at[wid, e1], idx1, is1).wait()
            pltpu.async_copy(ones_v, acc_sh.at[idx1.at[0]], ss1, add=True)
            pltpu.make_async_copy(ones_v, acc_sh.at[idx0.at[0]], ss0).wait()
            pltpu.async_copy(dst_hbm.at[wid, e0 + 2], idx0, is0)
            pltpu.make_async_copy(ones_v, acc_sh.at[idx1.at[0]], ss1).wait()
            pltpu.async_copy(dst_hbm.at[wid, e1 + 2], idx1, is1)
            return carry

        lax.fori_loop(0, _CHD // 2 - 1, body, 0)
        pltpu.make_async_copy(dst_hbm.at[wid, 0], idx0, is0).wait()
        pltpu.sync_copy(ones_v, acc_sh.at[idx0.at[0]], add=True)
        pltpu.make_async_copy(dst_hbm.at[wid, 0], idx1, is1).wait()
        pltpu.sync_copy(ones_v, acc_sh.at[idx1.at[0]], add=True)
        plsc.subcore_barrier()
        pltpu.sync_copy(acc_sh.at[pl.ds(s * _RPS, _RPS)],
                        out_hbm.at[c].at[pl.ds(s * _RPS, _RPS)])

    @functools.partial(
        pl.kernel,
        mesh=mesh,
        out_type=jax.ShapeDtypeStruct((_NC, _NP, _D), jnp.float32),
        scratch_types=[
            pltpu.VMEM((2, _CBA), jnp.int32),
            pltpu.VMEM((2, _CBA), jnp.int32),
            pltpu.VMEM((2, _CBA), jnp.int32),
            pltpu.VMEM((_CBA, _D), jnp.float32),
            pltpu.VMEM((_CBA, _D), jnp.float32),
            pltpu.VMEM((_CBA, _D), jnp.float32),
            pltpu.VMEM_SHARED((_NP, _D), jnp.float32),
            pltpu.SemaphoreType.DMA,
            pltpu.SemaphoreType.DMA,
            pltpu.SemaphoreType.DMA,
            pltpu.SemaphoreType.DMA,
            pltpu.SemaphoreType.DMA,
            pltpu.SemaphoreType.DMA,
            pltpu.SemaphoreType.DMA,
            pltpu.SemaphoreType.DMA,
            pltpu.SemaphoreType.DMA,
        ],
    )
    def sc_aggregate(y_hbm, ei_hbm, zero_hbm, out_hbm,
                     idx0, idx1, idx2, rows0, rows1, rows2, acc_sh,
                     is0, is1, is2, gs0, gs1, gs2, ss0, ss1, ss2):
        c = lax.axis_index("c")
        s = lax.axis_index("s")
        wid = s * _NC + c
        pltpu.sync_copy(zero_hbm, acc_sh.at[pl.ds(s * _RPS, _RPS)])
        plsc.subcore_barrier()

        nch = lax.select(c == _FAST, _CHF, _CHS)
        pltpu.async_copy(ei_hbm.at[wid, 0], idx0, is0)
        pltpu.async_copy(ei_hbm.at[wid, 1], idx1, is1)
        pltpu.async_copy(ei_hbm.at[wid, 2], idx2, is2)
        pltpu.make_async_copy(ei_hbm.at[wid, 0], idx0, is0).wait()
        pltpu.async_copy(y_hbm.at[idx0.at[0]], rows0, gs0)
        pltpu.make_async_copy(ei_hbm.at[wid, 0], idx1, is1).wait()
        pltpu.async_copy(y_hbm.at[idx1.at[0]], rows1, gs1)

        def body(j, carry):
            e0 = 3 * j
            pltpu.make_async_copy(y_hbm.at[idx0.at[0]], rows0, gs0).wait()
            pltpu.async_copy(rows0, acc_sh.at[idx0.at[1]], ss0, add=True)
            pltpu.make_async_copy(ei_hbm.at[wid, 0], idx2, is2).wait()
            pltpu.async_copy(y_hbm.at[idx2.at[0]], rows2, gs2)
            pltpu.make_async_copy(rows0, acc_sh.at[idx0.at[1]], ss0).wait()
            pltpu.async_copy(ei_hbm.at[wid, e0 + 3], idx0, is0)
            pltpu.make_async_copy(y_hbm.at[idx1.at[0]], rows1, gs1).wait()
            pltpu.async_copy(rows1, acc_sh.at[idx1.at[1]], ss1, add=True)
            pltpu.make_async_copy(ei_hbm.at[wid, 0], idx0, is0).wait()
            pltpu.async_copy(y_hbm.at[idx0.at[0]], rows0, gs0)
            pltpu.make_async_copy(rows1, acc_sh.at[idx1.at[1]], ss1).wait()
            pltpu.async_copy(ei_hbm.at[wid, e0 + 4], idx1, is1)
            pltpu.make_async_copy(y_hbm.at[idx2.at[0]], rows2, gs2).wait()
            pltpu.async_copy(rows2, acc_sh.at[idx2.at[1]], ss2, add=True)
            pltpu.make_async_copy(ei_hbm.at[wid, 0], idx1, is1).wait()
            pltpu.async_copy(y_hbm.at[idx1.at[0]], rows1, gs1)
            pltpu.make_async_copy(rows2, acc_sh.at[idx2.at[1]], ss2).wait()
            pltpu.async_copy(ei_hbm.at[wid, e0 + 5], idx2, is2)
            return carry

        lax.fori_loop(0, nch // 3 - 1, body, 0)
        pltpu.make_async_copy(y_hbm.at[idx0.at[0]], rows0, gs0).wait()
        pltpu.sync_copy(rows0, acc_sh.at[idx0.at[1]], add=True)
        pltpu.make_async_copy(y_hbm.at[idx1.at[0]], rows1, gs1).wait()
        pltpu.sync_copy(rows1, acc_sh.at[idx1.at[1]], add=True)
        pltpu.make_async_copy(ei_hbm.at[wid, 0], idx2, is2).wait()
        pltpu.sync_copy(y_hbm.at[idx2.at[0]], rows2)
        pltpu.sync_copy(rows2, acc_sh.at[idx2.at[1]], add=True)
        plsc.subcore_barrier()
        pltpu.sync_copy(acc_sh.at[pl.ds(s * _RPS, _RPS)],
                        out_hbm.at[c].at[pl.ds(s * _RPS, _RPS)])

    return sc_degree, sc_aggregate


def _tc_matmul1(x_p, W):
    def body(x_ref, w_ref, xw_ref):
        xw_ref[...] = jnp.dot(x_ref[...], w_ref[...],
                              preferred_element_type=jnp.float32)

    return pl.pallas_call(
        body,
        grid=(_NP // _R,),
        in_specs=[
            pl.BlockSpec((_R, _D), lambda i: (i, 0)),
            pl.BlockSpec((_D, _D), lambda i: (0, 0)),
        ],
        out_specs=pl.BlockSpec((_R, _D), lambda i: (i, 0)),
        out_shape=jax.ShapeDtypeStruct((_NP, _D), jnp.float32),
    )(x_p, W)


def _tc_scale(degp, xw):
    def body(deg_ref, xw_ref, y_ref, dinv_ref):
        deg = deg_ref[0] + deg_ref[1] + 1.0
        dinv = lax.rsqrt(deg)
        y_ref[...] = dinv * xw_ref[...]
        dinv_ref[...] = dinv

    return pl.pallas_call(
        body,
        grid=(_NP // _R,),
        in_specs=[
            pl.BlockSpec((_NC, _R, _DW), lambda i: (0, i, 0)),
            pl.BlockSpec((_R, _D), lambda i: (i, 0)),
        ],
        out_specs=[
            pl.BlockSpec((_R, _D), lambda i: (i, 0)),
            pl.BlockSpec((_R, _DW), lambda i: (i, 0)),
        ],
        out_shape=[
            jax.ShapeDtypeStruct((_NP, _D), jnp.float32),
            jax.ShapeDtypeStruct((_NP, _DW), jnp.float32),
        ],
    )(degp, xw)


def _tc_mid(parts, xw1, dinv, b1, W2):
    def body(p_ref, xw_ref, dinv_ref, b_ref, w_ref, y_ref, xw2_ref):
        dv = dinv_ref[...]
        h = dv * (p_ref[0] + p_ref[1]) + (dv * dv) * xw_ref[...] + b_ref[...]
        h = jnp.maximum(h, 0.0)
        xw2 = jnp.dot(h, w_ref[...], preferred_element_type=jnp.float32)
        y_ref[...] = dv * xw2
        xw2_ref[...] = xw2

    return pl.pallas_call(
        body,
        grid=(_NP // _R,),
        in_specs=[
            pl.BlockSpec((_NC, _R, _D), lambda i: (0, i, 0)),
            pl.BlockSpec((_R, _D), lambda i: (i, 0)),
            pl.BlockSpec((_R, _DW), lambda i: (i, 0)),
            pl.BlockSpec((1, _D), lambda i: (0, 0)),
            pl.BlockSpec((_D, _D), lambda i: (0, 0)),
        ],
        out_specs=[
            pl.BlockSpec((_R, _D), lambda i: (i, 0)),
            pl.BlockSpec((_R, _D), lambda i: (i, 0)),
        ],
        out_shape=[
            jax.ShapeDtypeStruct((_NP, _D), jnp.float32),
            jax.ShapeDtypeStruct((_NP, _D), jnp.float32),
        ],
    )(parts, xw1, dinv, b1, W2)


def _tc_final(parts, xw2, dinv, b2):
    def body(p_ref, xw_ref, dinv_ref, b_ref, o_ref):
        dv = dinv_ref[...]
        o_ref[...] = dv * (p_ref[0] + p_ref[1]) + (dv * dv) * xw_ref[...] + b_ref[...]

    return pl.pallas_call(
        body,
        grid=(_NP // _R,),
        in_specs=[
            pl.BlockSpec((_NC, _R, _D), lambda i: (0, i, 0)),
            pl.BlockSpec((_R, _D), lambda i: (i, 0)),
            pl.BlockSpec((_R, _DW), lambda i: (i, 0)),
            pl.BlockSpec((1, _D), lambda i: (0, 0)),
        ],
        out_specs=pl.BlockSpec((_R, _D), lambda i: (i, 0)),
        out_shape=jax.ShapeDtypeStruct((_NP, _D), jnp.float32),
    )(parts, xw2, dinv, b2)


def _split_edges(v):
    esr = (_E - (_NW // _NC) * _EF) // (_NW // _NC)
    slow_base = (_NW // _NC) * _EF
    rows = []
    for wid in range(_NW):
        k = wid // _NC
        if wid % _NC == _FAST:
            r = v[k * _EF:(k + 1) * _EF]
        else:
            r = v[slow_base + k * esr: slow_base + (k + 1) * esr]
            r = jnp.concatenate([r, jnp.full((_EF - esr,), _N, jnp.int32)])
        rows.append(r)
    return jnp.stack(rows).reshape(_NW, _CHF, _CBA)


def kernel(x, edge_index, W1, b1, W2, b2):
    src = edge_index[0].astype(jnp.int32)
    dst = edge_index[1].astype(jnp.int32)
    pad = jnp.full((_EP - _E,), _N, dtype=jnp.int32)
    dst_deg = jnp.concatenate([dst, pad]).reshape(_NW, _CHD, 1, _CBD)
    ei_a = jnp.stack([_split_edges(src), _split_edges(dst)], axis=2)
    x_p = jnp.pad(x, ((0, _NP - _N), (0, 0)))
    ones_dw = jnp.ones((_CBD, _DW), jnp.float32)
    zero_d = jnp.zeros((_RPS, _D), jnp.float32)

    sc_degree, sc_aggregate = _sc_kernels()
    degp = sc_degree(dst_deg, ones_dw, zero_d)
    xw1 = _tc_matmul1(x_p, W1)
    y1, dinv = _tc_scale(degp, xw1)
    p1 = sc_aggregate(y1, ei_a, zero_d)
    y2, xw2 = _tc_mid(p1, xw1, dinv, b1.reshape(1, _D), W2)
    p2 = sc_aggregate(y2, ei_a, zero_d)
    out = _tc_final(p2, xw2, dinv, b2.reshape(1, _D))
    return out[:_N]


# speedup vs baseline: 1.3551x; 1.0153x over previous
import functools

import jax
import jax.numpy as jnp
from jax import lax
from jax.experimental import pallas as pl
from jax.experimental.pallas import tpu as pltpu
from jax.experimental.pallas import tpu_sc as plsc

_N = 10000
_D = 128
_E = 320000
_NP = 10240
_NC = 2
_NS = 16
_NW = _NC * _NS
_CBD = 64
_CHD = 158
_EP = _NW * _CBD * _CHD
_RPS = _NP // _NS
_DW = 128
_R = 1024

_CBA = 72
_CHF = 184
_CHS = 96
_FAST = 0
_EF = _CHF * _CBA
_ES = _CHS * _CBA

@functools.cache
def _sc_kernels():
    mesh = plsc.VectorSubcoreMesh(core_axis_name="c", subcore_axis_name="s")

    @functools.partial(
        pl.kernel,
        mesh=mesh,
        out_type=jax.ShapeDtypeStruct((_NC, _NP, _DW), jnp.float32),
        scratch_types=[
            pltpu.VMEM((1, _CBD), jnp.int32),
            pltpu.VMEM((1, _CBD), jnp.int32),
            pltpu.VMEM((_CBD, _DW), jnp.float32),
            pltpu.VMEM_SHARED((_NP, _DW), jnp.float32),
            pltpu.SemaphoreType.DMA,
            pltpu.SemaphoreType.DMA,
            pltpu.SemaphoreType.DMA,
            pltpu.SemaphoreType.DMA,
        ],
    )
    def sc_degree(dst_hbm, ones_hbm, zero_hbm, out_hbm,
                  idx0, idx1, ones_v, acc_sh, is0, is1, ss0, ss1):
        c = lax.axis_index("c")
        s = lax.axis_index("s")
        wid = s * _NC + c
        pltpu.sync_copy(ones_hbm, ones_v)
        pltpu.sync_copy(zero_hbm, acc_sh.at[pl.ds(s * _RPS, _RPS)])
        plsc.subcore_barrier()

        pltpu.async_copy(dst_hbm.at[wid, 0], idx0, is0)
        pltpu.async_copy(dst_hbm.at[wid, 1], idx1, is1)

        def body(j, carry):
            e0 = 2 * j
            e1 = 2 * j + 1
            pltpu.make_async_copy(dst_hbm.at[wid, e0], idx0, is0).wait()
            pltpu.async_copy(ones_v, acc_sh.at[idx0.at[0]], ss0, add=True)
            pltpu.make_async_copy(dst_hbm.at[wid, e1], idx1, is1).wait()
            pltpu.async_copy(ones_v, acc_sh.at[idx1.at[0]], ss1, add=True)
            pltpu.make_async_copy(ones_v, acc_sh.at[idx0.at[0]], ss0).wait()
            pltpu.async_copy(dst_hbm.at[wid, e0 + 2], idx0, is0)
            pltpu.make_async_copy(ones_v, acc_sh.at[idx1.at[0]], ss1).wait()
            pltpu.async_copy(dst_hbm.at[wid, e1 + 2], idx1, is1)
            return carry

        lax.fori_loop(0, _CHD // 2 - 1, body, 0)
        pltpu.make_async_copy(dst_hbm.at[wid, 0], idx0, is0).wait()
        pltpu.sync_copy(ones_v, acc_sh.at[idx0.at[0]], add=True)
        pltpu.make_async_copy(dst_hbm.at[wid, 0], idx1, is1).wait()
        pltpu.sync_copy(ones_v, acc_sh.at[idx1.at[0]], add=True)
        plsc.subcore_barrier()
        pltpu.sync_copy(acc_sh.at[pl.ds(s * _RPS, _RPS)],
                        out_hbm.at[c].at[pl.ds(s * _RPS, _RPS)])

    @functools.partial(
        pl.kernel,
        mesh=mesh,
        out_type=jax.ShapeDtypeStruct((_NC, _NP, _D), jnp.float32),
        scratch_types=(
            [pltpu.VMEM((2, _CBA), jnp.int32)] * 4
            + [pltpu.VMEM((_CBA, _D), jnp.float32)] * 4
            + [pltpu.VMEM_SHARED((_NP, _D), jnp.float32)]
            + [pltpu.SemaphoreType.DMA] * 12
        ),
    )
    def sc_aggregate(y_hbm, ei_hbm, zero_hbm, out_hbm,
                     idx0, idx1, idx2, idx3, rows0, rows1, rows2, rows3,
                     acc_sh, is0, is1, is2, is3, gs0, gs1, gs2, gs3,
                     ss0, ss1, ss2, ss3):
        c = lax.axis_index("c")
        s = lax.axis_index("s")
        wid = s * _NC + c
        pltpu.sync_copy(zero_hbm, acc_sh.at[pl.ds(s * _RPS, _RPS)])
        plsc.subcore_barrier()

        nch = lax.select(c == _FAST, _CHF, _CHS)
        pltpu.async_copy(ei_hbm.at[wid, 0], idx0, is0)
        pltpu.async_copy(ei_hbm.at[wid, 1], idx1, is1)
        pltpu.async_copy(ei_hbm.at[wid, 2], idx2, is2)
        pltpu.async_copy(ei_hbm.at[wid, 3], idx3, is3)
        pltpu.make_async_copy(ei_hbm.at[wid, 0], idx0, is0).wait()
        pltpu.async_copy(y_hbm.at[idx0.at[0]], rows0, gs0)
        pltpu.make_async_copy(ei_hbm.at[wid, 0], idx1, is1).wait()
        pltpu.async_copy(y_hbm.at[idx1.at[0]], rows1, gs1)
        pltpu.make_async_copy(ei_hbm.at[wid, 0], idx2, is2).wait()
        pltpu.async_copy(y_hbm.at[idx2.at[0]], rows2, gs2)

        def body(j, carry):
            e0 = 4 * j
            pltpu.make_async_copy(y_hbm.at[idx0.at[0]], rows0, gs0).wait()
            pltpu.async_copy(rows0, acc_sh.at[idx0.at[1]], ss0, add=True)
            pltpu.make_async_copy(ei_hbm.at[wid, 0], idx3, is3).wait()
            pltpu.async_copy(y_hbm.at[idx3.at[0]], rows3, gs3)
            pltpu.make_async_copy(rows0, acc_sh.at[idx0.at[1]], ss0).wait()
            pltpu.async_copy(ei_hbm.at[wid, e0 + 4], idx0, is0)
            pltpu.make_async_copy(y_hbm.at[idx1.at[0]], rows1, gs1).wait()
            pltpu.async_copy(rows1, acc_sh.at[idx1.at[1]], ss1, add=True)
            pltpu.make_async_copy(ei_hbm.at[wid, 0], idx0, is0).wait()
            pltpu.async_copy(y_hbm.at[idx0.at[0]], rows0, gs0)
            pltpu.make_async_copy(rows1, acc_sh.at[idx1.at[1]], ss1).wait()
            pltpu.async_copy(ei_hbm.at[wid, e0 + 5], idx1, is1)
            pltpu.make_async_copy(y_hbm.at[idx2.at[0]], rows2, gs2).wait()
            pltpu.async_copy(rows2, acc_sh.at[idx2.at[1]], ss2, add=True)
            pltpu.make_async_copy(ei_hbm.at[wid, 0], idx1, is1).wait()
            pltpu.async_copy(y_hbm.at[idx1.at[0]], rows1, gs1)
            pltpu.make_async_copy(rows2, acc_sh.at[idx2.at[1]], ss2).wait()
            pltpu.async_copy(ei_hbm.at[wid, e0 + 6], idx2, is2)
            pltpu.make_async_copy(y_hbm.at[idx3.at[0]], rows3, gs3).wait()
            pltpu.async_copy(rows3, acc_sh.at[idx3.at[1]], ss3, add=True)
            pltpu.make_async_copy(ei_hbm.at[wid, 0], idx2, is2).wait()
            pltpu.async_copy(y_hbm.at[idx2.at[0]], rows2, gs2)
            pltpu.make_async_copy(rows3, acc_sh.at[idx3.at[1]], ss3).wait()
            pltpu.async_copy(ei_hbm.at[wid, e0 + 7], idx3, is3)
            return carry

        lax.fori_loop(0, nch // 4 - 1, body, 0)
        pltpu.make_async_copy(y_hbm.at[idx0.at[0]], rows0, gs0).wait()
        pltpu.sync_copy(rows0, acc_sh.at[idx0.at[1]], add=True)
        pltpu.make_async_copy(y_hbm.at[idx1.at[0]], rows1, gs1).wait()
        pltpu.sync_copy(rows1, acc_sh.at[idx1.at[1]], add=True)
        pltpu.make_async_copy(y_hbm.at[idx2.at[0]], rows2, gs2).wait()
        pltpu.sync_copy(rows2, acc_sh.at[idx2.at[1]], add=True)
        pltpu.make_async_copy(ei_hbm.at[wid, 0], idx3, is3).wait()
        pltpu.sync_copy(y_hbm.at[idx3.at[0]], rows3)
        pltpu.sync_copy(rows3, acc_sh.at[idx3.at[1]], add=True)
        plsc.subcore_barrier()
        pltpu.sync_copy(acc_sh.at[pl.ds(s * _RPS, _RPS)],
                        out_hbm.at[c].at[pl.ds(s * _RPS, _RPS)])

    return sc_degree, sc_aggregate


def _tc_matmul1(x_p, W):
    def body(x_ref, w_ref, xw_ref):
        xw_ref[...] = jnp.dot(x_ref[...], w_ref[...],
                              preferred_element_type=jnp.float32)

    return pl.pallas_call(
        body,
        grid=(_NP // _R,),
        in_specs=[
            pl.BlockSpec((_R, _D), lambda i: (i, 0)),
            pl.BlockSpec((_D, _D), lambda i: (0, 0)),
        ],
        out_specs=pl.BlockSpec((_R, _D), lambda i: (i, 0)),
        out_shape=jax.ShapeDtypeStruct((_NP, _D), jnp.float32),
    )(x_p, W)


def _tc_scale(degp, xw):
    def body(deg_ref, xw_ref, y_ref, dinv_ref):
        deg = deg_ref[0] + deg_ref[1] + 1.0
        dinv = lax.rsqrt(deg)
        y_ref[...] = dinv * xw_ref[...]
        dinv_ref[...] = dinv

    return pl.pallas_call(
        body,
        grid=(_NP // _R,),
        in_specs=[
            pl.BlockSpec((_NC, _R, _DW), lambda i: (0, i, 0)),
            pl.BlockSpec((_R, _D), lambda i: (i, 0)),
        ],
        out_specs=[
            pl.BlockSpec((_R, _D), lambda i: (i, 0)),
            pl.BlockSpec((_R, _DW), lambda i: (i, 0)),
        ],
        out_shape=[
            jax.ShapeDtypeStruct((_NP, _D), jnp.float32),
            jax.ShapeDtypeStruct((_NP, _DW), jnp.float32),
        ],
    )(degp, xw)


def _tc_mid(parts, xw1, dinv, b1, W2):
    def body(p_ref, xw_ref, dinv_ref, b_ref, w_ref, y_ref, xw2_ref):
        dv = dinv_ref[...]
        h = dv * (p_ref[0] + p_ref[1]) + (dv * dv) * xw_ref[...] + b_ref[...]
        h = jnp.maximum(h, 0.0)
        xw2 = jnp.dot(h, w_ref[...], preferred_element_type=jnp.float32)
        y_ref[...] = dv * xw2
        xw2_ref[...] = xw2

    return pl.pallas_call(
        body,
        grid=(_NP // _R,),
        in_specs=[
            pl.BlockSpec((_NC, _R, _D), lambda i: (0, i, 0)),
            pl.BlockSpec((_R, _D), lambda i: (i, 0)),
            pl.BlockSpec((_R, _DW), lambda i: (i, 0)),
            pl.BlockSpec((1, _D), lambda i: (0, 0)),
            pl.BlockSpec((_D, _D), lambda i: (0, 0)),
        ],
        out_specs=[
            pl.BlockSpec((_R, _D), lambda i: (i, 0)),
            pl.BlockSpec((_R, _D), lambda i: (i, 0)),
        ],
        out_shape=[
            jax.ShapeDtypeStruct((_NP, _D), jnp.float32),
            jax.ShapeDtypeStruct((_NP, _D), jnp.float32),
        ],
    )(parts, xw1, dinv, b1, W2)


def _tc_final(parts, xw2, dinv, b2):
    def body(p_ref, xw_ref, dinv_ref, b_ref, o_ref):
        dv = dinv_ref[...]
        o_ref[...] = dv * (p_ref[0] + p_ref[1]) + (dv * dv) * xw_ref[...] + b_ref[...]

    return pl.pallas_call(
        body,
        grid=(_NP // _R,),
        in_specs=[
            pl.BlockSpec((_NC, _R, _D), lambda i: (0, i, 0)),
            pl.BlockSpec((_R, _D), lambda i: (i, 0)),
            pl.BlockSpec((_R, _DW), lambda i: (i, 0)),
            pl.BlockSpec((1, _D), lambda i: (0, 0)),
        ],
        out_specs=pl.BlockSpec((_R, _D), lambda i: (i, 0)),
        out_shape=jax.ShapeDtypeStruct((_NP, _D), jnp.float32),
    )(parts, xw2, dinv, b2)


def _split_edges(v):
    esr = (_E - (_NW // _NC) * _EF) // (_NW // _NC)
    slow_base = (_NW // _NC) * _EF
    rows = []
    for wid in range(_NW):
        k = wid // _NC
        if wid % _NC == _FAST:
            r = v[k * _EF:(k + 1) * _EF]
        else:
            r = v[slow_base + k * esr: slow_base + (k + 1) * esr]
            r = jnp.concatenate([r, jnp.full((_EF - esr,), _N, jnp.int32)])
        rows.append(r)
    return jnp.stack(rows).reshape(_NW, _CHF, _CBA)


def kernel(x, edge_index, W1, b1, W2, b2):
    src = edge_index[0].astype(jnp.int32)
    dst = edge_index[1].astype(jnp.int32)
    pad = jnp.full((_EP - _E,), _N, dtype=jnp.int32)
    dst_deg = jnp.concatenate([dst, pad]).reshape(_NW, _CHD, 1, _CBD)
    ei_a = jnp.stack([_split_edges(src), _split_edges(dst)], axis=2)
    x_p = jnp.pad(x, ((0, _NP - _N), (0, 0)))
    ones_dw = jnp.ones((_CBD, _DW), jnp.float32)
    zero_d = jnp.zeros((_RPS, _D), jnp.float32)

    sc_degree, sc_aggregate = _sc_kernels()
    degp = sc_degree(dst_deg, ones_dw, zero_d)
    xw1 = _tc_matmul1(x_p, W1)
    y1, dinv = _tc_scale(degp, xw1)
    p1 = sc_aggregate(y1, ei_a, zero_d)
    y2, xw2 = _tc_mid(p1, xw1, dinv, b1.reshape(1, _D), W2)
    p2 = sc_aggregate(y2, ei_a, zero_d)
    out = _tc_final(p2, xw2, dinv, b2.reshape(1, _D))
    return out[:_N]
